# baseline (speedup 1.0000x reference)
import functools

import jax
import jax.numpy as jnp
from jax import lax
from jax.experimental import pallas as pl
from jax.experimental.pallas import tpu as pltpu
from jax.experimental.pallas import tpu_sc as plsc

_N = 10000
_E = 320000
_F = 128
_NC = 2
_NS = 16
_NW = _NC * _NS
_EPW = _E // _NW
_C = 80
_NCHUNK = _EPW // _C
_RPT = 624
_TAIL0 = _NS * _RPT
_TAIL = _N - _TAIL0


_NPAIR = (_NCHUNK - 1) // 2


def _segsum_body(h_hbm, src_hbm, dst_hbm, zeros_hbm, out_hbm,
                 src_v, dst_v, rows_v, acc_sh, gsems, ssems, isem):
    c = lax.axis_index("c")
    s = lax.axis_index("s")
    wid = s * _NC + c
    r0 = s * _RPT
    pltpu.sync_copy(zeros_hbm.at[pl.ds(r0, _RPT)], acc_sh.at[pl.ds(r0, _RPT)])

    @pl.when(s == _NS - 1)
    def _zero_tail():
        pltpu.sync_copy(zeros_hbm.at[pl.ds(_TAIL0, _TAIL)],
                        acc_sh.at[pl.ds(_TAIL0, _TAIL)])

    plsc.subcore_barrier()

    def _idx_fetch(g, p, j):
        pltpu.async_copy(src_hbm.at[wid, g], src_v.at[p, j], isem)
        pltpu.async_copy(dst_hbm.at[wid, g], dst_v.at[p, j], isem)

    def _idx_wait(g, p, j):
        pltpu.make_async_copy(src_hbm.at[wid, g], src_v.at[p, j],
                              isem).wait()
        pltpu.make_async_copy(dst_hbm.at[wid, g], dst_v.at[p, j],
                              isem).wait()

    def _gather(p, j, b):
        return pltpu.async_copy(h_hbm.at[src_v.at[p, j]], rows_v.at[b],
                                gsems[b])

    def _gather_wait(p, j, b):
        pltpu.make_async_copy(h_hbm.at[src_v.at[p, j]], rows_v.at[b],
                              gsems[b]).wait()

    def _scatter(p, j, b):
        return pltpu.async_copy(rows_v.at[b], acc_sh.at[dst_v.at[p, j]],
                                ssems[b], add=True)

    def _scatter_wait(p, j, b):
        pltpu.make_async_copy(rows_v.at[b], acc_sh.at[dst_v.at[p, j]],
                              ssems[b]).wait()

    _idx_fetch(0, 0, 0)
    _idx_fetch(1, 0, 1)
    _idx_wait(0, 0, 0)
    _idx_wait(1, 0, 1)
    _gather(0, 0, 0)

    def pair(w, carry):
        g = 2 * w
        p = lax.rem(w, 2)
        q = 1 - p
        _idx_fetch(g + 2, q, 0)

        @pl.when(w < _NPAIR - 1)
        def _fetch2():
            _idx_fetch(g + 3, q, 1)

        @pl.when(w > 0)
        def _reclaim():
            _scatter_wait(p, 1, 1)

        _gather(p, 1, 1)
        _gather_wait(p, 0, 0)
        _scatter(p, 0, 0)
        _gather_wait(p, 1, 1)
        _scatter(p, 1, 1)
        _scatter_wait(p, 0, 0)
        _idx_wait(g + 2, q, 0)

        @pl.when(w < _NPAIR - 1)
        def _wait2():
            _idx_wait(g + 3, q, 1)

        _gather(q, 0, 0)
        return carry

    lax.fori_loop(0, _NPAIR, pair, 0)
    pfin = _NPAIR % 2
    _gather_wait(pfin, 0, 0)
    _scatter(pfin, 0, 0)
    _scatter_wait(1 - pfin, 1, 1)
    _scatter_wait(pfin, 0, 0)
    plsc.subcore_barrier()
    pltpu.sync_copy(acc_sh.at[pl.ds(r0, _RPT)], out_hbm.at[c, pl.ds(r0, _RPT)])

    @pl.when(s == _NS - 1)
    def _dump_tail():
        pltpu.sync_copy(acc_sh.at[pl.ds(_TAIL0, _TAIL)],
                        out_hbm.at[c, pl.ds(_TAIL0, _TAIL)])


_segsum = pl.kernel(
    _segsum_body,
    out_type=jax.ShapeDtypeStruct((_NC, _N, _F), jnp.float32),
    mesh=plsc.VectorSubcoreMesh(core_axis_name="c", subcore_axis_name="s"),
    scratch_types=[
        pltpu.VMEM((2, 2, _C), jnp.int32),
        pltpu.VMEM((2, 2, _C), jnp.int32),
        pltpu.VMEM((2, _C, _F), jnp.float32),
        pltpu.VMEM_SHARED((_N, _F), jnp.float32),
        [pltpu.SemaphoreType.DMA] * 2,
        [pltpu.SemaphoreType.DMA] * 2,
        pltpu.SemaphoreType.DMA,
    ],
)


def _dense_body(h_ref, a_ref, w1t_ref, b1_ref, w2t_ref, b2_ref,
                g_ref, be_ref, out_ref, t_scr, s_scr, *, last):
    ph = pl.program_id(0)
    i = pl.program_id(1)

    @pl.when(ph == 0)
    def _compute():
        z = h_ref[...] + a_ref[0] + a_ref[1]
        u = jnp.maximum(
            jnp.dot(z, w1t_ref[...], preferred_element_type=jnp.float32)
            + b1_ref[...], 0.0)
        t = (jnp.dot(u, w2t_ref[...], preferred_element_type=jnp.float32)
             + b2_ref[...])
        t_scr[pl.ds(i * _BD, _BD), :] = t
        out_ref[...] = t
        ps = jnp.concatenate(
            [jnp.sum(t, axis=0, keepdims=True),
             jnp.sum(t * t, axis=0, keepdims=True)], axis=0)

        @pl.when(i == 0)
        def _init():
            s_scr[...] = ps

        @pl.when(i > 0)
        def _accum():
            s_scr[...] += ps

    @pl.when(ph == 1)
    def _normalize():
        mean = s_scr[0:1] * (1.0 / _N)
        var = s_scr[1:2] * (1.0 / _N) - mean * mean
        t = t_scr[pl.ds(i * _BD, _BD), :]
        y = (t - mean) * lax.rsqrt(var + 1e-5) * g_ref[...] + be_ref[...]
        if not last:
            y = jnp.maximum(y, 0.0)
        out_ref[...] = y


_BD = 1000


def _dense_layer(h, acc, w1, b1, w2, b2, gamma, beta, last):
    nb = _N // _BD
    fn = pl.pallas_call(
        functools.partial(_dense_body, last=last),
        grid=(2, nb),
        in_specs=[
            pl.BlockSpec((_BD, _F), lambda ph, i: (i, 0)),
            pl.BlockSpec((2, _BD, _F), lambda ph, i: (0, i, 0)),
            pl.BlockSpec((_F, _F), lambda ph, i: (0, 0)),
            pl.BlockSpec((1, _F), lambda ph, i: (0, 0)),
            pl.BlockSpec((_F, _F), lambda ph, i: (0, 0)),
            pl.BlockSpec((1, _F), lambda ph, i: (0, 0)),
            pl.BlockSpec((1, _F), lambda ph, i: (0, 0)),
            pl.BlockSpec((1, _F), lambda ph, i: (0, 0)),
        ],
        out_specs=pl.BlockSpec((_BD, _F), lambda ph, i: (i, 0)),
        out_shape=jax.ShapeDtypeStruct((_N, _F), jnp.float32),
        scratch_shapes=[
            pltpu.VMEM((_N, _F), jnp.float32),
            pltpu.VMEM((2, _F), jnp.float32),
        ],
    )
    return fn(h, acc, w1.T, b1.reshape(1, _F), w2.T,
              b2.reshape(1, _F), gamma.reshape(1, _F), beta.reshape(1, _F))


def kernel(x, edge_index, edge_attr, W1_0, b1_0, W2_0, b2_0,
           W1_1, b1_1, W2_1, b2_1, gamma_0, beta_0, gamma_1, beta_1):
    src = edge_index[0].reshape(_NW, _NCHUNK, _C)
    dst = edge_index[1].reshape(_NW, _NCHUNK, _C)
    zeros = jnp.zeros((_N, _F), dtype=jnp.float32)
    acc0 = _segsum(x, src, dst, zeros)
    h1 = _dense_layer(x, acc0, W1_0, b1_0, W2_0, b2_0, gamma_0, beta_0,
                      last=False)
    acc1 = _segsum(h1, src, dst, zeros)
    h2 = _dense_layer(h1, acc1, W1_1, b1_1, W2_1, b2_1, gamma_1, beta_1,
                      last=True)
    return h2[:, None, :]


# speedup vs baseline: 1.0584x; 1.0584x over previous
import functools

import jax
import jax.numpy as jnp
from jax import lax
from jax.experimental import pallas as pl
from jax.experimental.pallas import tpu as pltpu
from jax.experimental.pallas import tpu_sc as plsc

_N = 10000
_E = 320000
_F = 128
_NC = 2
_NS = 16
_NW = _NC * _NS
_EPW = _E // _NW
_C = 80
_NCHUNK = _EPW // _C
_RPT = 624
_TAIL0 = _NS * _RPT
_TAIL = _N - _TAIL0


_NPAIR = (_NCHUNK - 1) // 2


def _segsum_body(h_hbm, src_hbm, dst_hbm, zeros_hbm, out_hbm,
                 src_v, dst_v, rows_v, acc_sh, gsems, ssems, isem):
    c = lax.axis_index("c")
    s = lax.axis_index("s")
    wid = s * _NC + c
    r0 = s * _RPT
    pltpu.sync_copy(zeros_hbm.at[pl.ds(r0, _RPT)], acc_sh.at[pl.ds(r0, _RPT)])

    @pl.when(s == _NS - 1)
    def _zero_tail():
        pltpu.sync_copy(zeros_hbm.at[pl.ds(_TAIL0, _TAIL)],
                        acc_sh.at[pl.ds(_TAIL0, _TAIL)])

    plsc.subcore_barrier()

    def _idx_fetch(g, p, j):
        pltpu.async_copy(src_hbm.at[wid, g], src_v.at[p, j], isem)
        pltpu.async_copy(dst_hbm.at[wid, g], dst_v.at[p, j], isem)

    def _idx_wait(g, p, j):
        pltpu.make_async_copy(src_hbm.at[wid, g], src_v.at[p, j],
                              isem).wait()
        pltpu.make_async_copy(dst_hbm.at[wid, g], dst_v.at[p, j],
                              isem).wait()

    def _gather(p, j, b):
        return pltpu.async_copy(h_hbm.at[src_v.at[p, j]], rows_v.at[b],
                                gsems[b])

    def _gather_wait(p, j, b):
        pltpu.make_async_copy(h_hbm.at[src_v.at[p, j]], rows_v.at[b],
                              gsems[b]).wait()

    def _scatter(p, j, b):
        return pltpu.async_copy(rows_v.at[b], acc_sh.at[dst_v.at[p, j]],
                                ssems[b], add=True)

    def _scatter_wait(p, j, b):
        pltpu.make_async_copy(rows_v.at[b], acc_sh.at[dst_v.at[p, j]],
                              ssems[b]).wait()

    _idx_fetch(0, 0, 0)
    _idx_fetch(1, 0, 1)
    _idx_wait(0, 0, 0)
    _idx_wait(1, 0, 1)
    _gather(0, 0, 0)

    def pair(w, carry):
        g = 2 * w
        p = lax.rem(w, 2)
        q = 1 - p
        _idx_fetch(g + 2, q, 0)

        @pl.when(w < _NPAIR - 1)
        def _fetch2():
            _idx_fetch(g + 3, q, 1)

        @pl.when(w > 0)
        def _reclaim():
            _scatter_wait(p, 1, 1)

        _gather(p, 1, 1)
        _gather_wait(p, 0, 0)
        _scatter(p, 0, 0)
        _gather_wait(p, 1, 1)
        _scatter(p, 1, 1)
        _scatter_wait(p, 0, 0)
        _idx_wait(g + 2, q, 0)

        @pl.when(w < _NPAIR - 1)
        def _wait2():
            _idx_wait(g + 3, q, 1)

        _gather(q, 0, 0)
        return carry

    lax.fori_loop(0, _NPAIR, pair, 0)
    pfin = _NPAIR % 2
    _gather_wait(pfin, 0, 0)
    _scatter(pfin, 0, 0)
    _scatter_wait(1 - pfin, 1, 1)
    _scatter_wait(pfin, 0, 0)
    plsc.subcore_barrier()
    pltpu.sync_copy(acc_sh.at[pl.ds(r0, _RPT)], out_hbm.at[c, pl.ds(r0, _RPT)])

    @pl.when(s == _NS - 1)
    def _dump_tail():
        pltpu.sync_copy(acc_sh.at[pl.ds(_TAIL0, _TAIL)],
                        out_hbm.at[c, pl.ds(_TAIL0, _TAIL)])


_segsum = pl.kernel(
    _segsum_body,
    out_type=jax.ShapeDtypeStruct((_NC, _N, _F), jnp.float32),
    mesh=plsc.VectorSubcoreMesh(core_axis_name="c", subcore_axis_name="s"),
    scratch_types=[
        pltpu.VMEM((2, 2, _C), jnp.int32),
        pltpu.VMEM((2, 2, _C), jnp.int32),
        pltpu.VMEM((2, _C, _F), jnp.float32),
        pltpu.VMEM_SHARED((_N, _F), jnp.float32),
        [pltpu.SemaphoreType.DMA] * 2,
        [pltpu.SemaphoreType.DMA] * 2,
        pltpu.SemaphoreType.DMA,
    ],
)


def _dense_body(h_ref, a_ref, w1t_ref, b1_ref, w2t_ref, b2_ref,
                g_ref, be_ref, out_ref, *, last):
    z = h_ref[...] + a_ref[0] + a_ref[1]
    u = jnp.maximum(
        jnp.dot(z, w1t_ref[...], preferred_element_type=jnp.float32)
        + b1_ref[...], 0.0)
    t = (jnp.dot(u, w2t_ref[...], preferred_element_type=jnp.float32)
         + b2_ref[...])
    mean = jnp.mean(t, axis=0, keepdims=True)
    d = t - mean
    var = jnp.mean(d * d, axis=0, keepdims=True)
    y = d * lax.rsqrt(var + 1e-5) * g_ref[...] + be_ref[...]
    if not last:
        y = jnp.maximum(y, 0.0)
    out_ref[...] = y


def _dense_layer(h, acc, w1, b1, w2, b2, gamma, beta, last):
    fn = pl.pallas_call(
        functools.partial(_dense_body, last=last),
        out_shape=jax.ShapeDtypeStruct((_N, _F), jnp.float32),
    )
    return fn(h, acc, w1.T, b1.reshape(1, _F), w2.T,
              b2.reshape(1, _F), gamma.reshape(1, _F), beta.reshape(1, _F))


def kernel(x, edge_index, edge_attr, W1_0, b1_0, W2_0, b2_0,
           W1_1, b1_1, W2_1, b2_1, gamma_0, beta_0, gamma_1, beta_1):
    src = edge_index[0].reshape(_NW, _NCHUNK, _C)
    dst = edge_index[1].reshape(_NW, _NCHUNK, _C)
    zeros = jnp.zeros((_N, _F), dtype=jnp.float32)
    acc0 = _segsum(x, src, dst, zeros)
    h1 = _dense_layer(x, acc0, W1_0, b1_0, W2_0, b2_0, gamma_0, beta_0,
                      last=False)
    acc1 = _segsum(h1, src, dst, zeros)
    h2 = _dense_layer(h1, acc1, W1_1, b1_1, W2_1, b2_1, gamma_1, beta_1,
                      last=True)
    return h2[:, None, :]


# speedup vs baseline: 1.0585x; 1.0001x over previous
import functools

import jax
import jax.numpy as jnp
from jax import lax
from jax.experimental import pallas as pl
from jax.experimental.pallas import tpu as pltpu
from jax.experimental.pallas import tpu_sc as plsc

_N = 10000
_E = 320000
_F = 128
_NC = 2
_NS = 16
_NW = _NC * _NS
_EPW = _E // _NW
_C = 80
_NCHUNK = _EPW // _C
_RPT = 624
_TAIL0 = _NS * _RPT
_TAIL = _N - _TAIL0


_NPAIR = (_NCHUNK - 1) // 2


def _segsum_body(h_hbm, src_hbm, dst_hbm, zeros_hbm, out_hbm,
                 src_v, dst_v, rows_v, acc_sh, gsems, ssems, isem):
    c = lax.axis_index("c")
    s = lax.axis_index("s")
    wid = s * _NC + c
    r0 = s * _RPT
    pltpu.sync_copy(zeros_hbm.at[pl.ds(r0, _RPT)], acc_sh.at[pl.ds(r0, _RPT)])

    @pl.when(s == _NS - 1)
    def _zero_tail():
        pltpu.sync_copy(zeros_hbm.at[pl.ds(_TAIL0, _TAIL)],
                        acc_sh.at[pl.ds(_TAIL0, _TAIL)])

    plsc.subcore_barrier()

    def _idx_fetch(g, p, j):
        pltpu.async_copy(src_hbm.at[wid, g], src_v.at[p, j], isem)
        pltpu.async_copy(dst_hbm.at[wid, g], dst_v.at[p, j], isem)

    def _idx_wait(g, p, j):
        pltpu.make_async_copy(src_hbm.at[wid, g], src_v.at[p, j],
                              isem).wait()
        pltpu.make_async_copy(dst_hbm.at[wid, g], dst_v.at[p, j],
                              isem).wait()

    def _gather(p, j, b):
        return pltpu.async_copy(h_hbm.at[src_v.at[p, j]], rows_v.at[b],
                                gsems[b])

    def _gather_wait(p, j, b):
        pltpu.make_async_copy(h_hbm.at[src_v.at[p, j]], rows_v.at[b],
                              gsems[b]).wait()

    def _scatter(p, j, b):
        return pltpu.async_copy(rows_v.at[b], acc_sh.at[dst_v.at[p, j]],
                                ssems[b], add=True)

    def _scatter_wait(p, j, b):
        pltpu.make_async_copy(rows_v.at[b], acc_sh.at[dst_v.at[p, j]],
                              ssems[b]).wait()

    _idx_fetch(0, 0, 0)
    _idx_fetch(1, 0, 1)
    _idx_wait(0, 0, 0)
    _idx_wait(1, 0, 1)
    _gather(0, 0, 0)

    def pair(w, carry):
        g = 2 * w
        p = lax.rem(w, 2)
        q = 1 - p
        _idx_fetch(g + 2, q, 0)

        @pl.when(w < _NPAIR - 1)
        def _fetch2():
            _idx_fetch(g + 3, q, 1)

        @pl.when(w > 0)
        def _reclaim():
            _scatter_wait(p, 1, 1)

        _gather(p, 1, 1)
        _gather_wait(p, 0, 0)
        _scatter(p, 0, 0)
        _gather_wait(p, 1, 1)
        _scatter(p, 1, 1)
        _scatter_wait(p, 0, 0)
        _idx_wait(g + 2, q, 0)

        @pl.when(w < _NPAIR - 1)
        def _wait2():
            _idx_wait(g + 3, q, 1)

        _gather(q, 0, 0)
        return carry

    lax.fori_loop(0, _NPAIR, pair, 0)
    pfin = _NPAIR % 2
    _gather_wait(pfin, 0, 0)
    _scatter(pfin, 0, 0)
    _scatter_wait(1 - pfin, 1, 1)
    _scatter_wait(pfin, 0, 0)
    plsc.subcore_barrier()
    pltpu.sync_copy(acc_sh.at[pl.ds(r0, _RPT)], out_hbm.at[c, pl.ds(r0, _RPT)])

    @pl.when(s == _NS - 1)
    def _dump_tail():
        pltpu.sync_copy(acc_sh.at[pl.ds(_TAIL0, _TAIL)],
                        out_hbm.at[c, pl.ds(_TAIL0, _TAIL)])


_segsum = pl.kernel(
    _segsum_body,
    out_type=jax.ShapeDtypeStruct((_NC, _N, _F), jnp.float32),
    mesh=plsc.VectorSubcoreMesh(core_axis_name="c", subcore_axis_name="s"),
    scratch_types=[
        pltpu.VMEM((2, 2, _C), jnp.int32),
        pltpu.VMEM((2, 2, _C), jnp.int32),
        pltpu.VMEM((2, _C, _F), jnp.float32),
        pltpu.VMEM_SHARED((_N, _F), jnp.float32),
        [pltpu.SemaphoreType.DMA] * 2,
        [pltpu.SemaphoreType.DMA] * 2,
        pltpu.SemaphoreType.DMA,
    ],
)


def _dense_body(h_ref, a_ref, w1t_ref, b1_ref, w2t_ref, b2_ref,
                g_ref, be_ref, out_ref, *, last):
    z = h_ref[...] + a_ref[0] + a_ref[1]
    dn = (((1,), (1,)), ((), ()))
    u = jnp.maximum(
        lax.dot_general(z, w1t_ref[...], dn,
                        preferred_element_type=jnp.float32)
        + b1_ref[...], 0.0)
    t = (lax.dot_general(u, w2t_ref[...], dn,
                         preferred_element_type=jnp.float32)
         + b2_ref[...])
    mean = jnp.mean(t, axis=0, keepdims=True)
    d = t - mean
    var = jnp.mean(d * d, axis=0, keepdims=True)
    y = d * lax.rsqrt(var + 1e-5) * g_ref[...] + be_ref[...]
    if not last:
        y = jnp.maximum(y, 0.0)
    out_ref[...] = y


def _dense_layer(h, acc, w1, b1, w2, b2, gamma, beta, last):
    fn = pl.pallas_call(
        functools.partial(_dense_body, last=last),
        out_shape=jax.ShapeDtypeStruct((_N, _F), jnp.float32),
    )
    return fn(h, acc, w1, b1.reshape(1, _F), w2,
              b2.reshape(1, _F), gamma.reshape(1, _F), beta.reshape(1, _F))


def kernel(x, edge_index, edge_attr, W1_0, b1_0, W2_0, b2_0,
           W1_1, b1_1, W2_1, b2_1, gamma_0, beta_0, gamma_1, beta_1):
    src = edge_index[0].reshape(_NW, _NCHUNK, _C)
    dst = edge_index[1].reshape(_NW, _NCHUNK, _C)
    zeros = jnp.zeros((_N, _F), dtype=jnp.float32)
    acc0 = _segsum(x, src, dst, zeros)
    h1 = _dense_layer(x, acc0, W1_0, b1_0, W2_0, b2_0, gamma_0, beta_0,
                      last=False)
    acc1 = _segsum(h1, src, dst, zeros)
    h2 = _dense_layer(h1, acc1, W1_1, b1_1, W2_1, b2_1, gamma_1, beta_1,
                      last=True)
    return h2[:, None, :]
